# baseline (speedup 1.0000x reference)
import functools

import jax
import jax.numpy as jnp
from jax import lax
from jax.experimental import pallas as pl
from jax.experimental.pallas import tpu as pltpu
from jax.experimental.pallas import tpu_sc as plsc

VOCAB = 49408
EMBED = 512
NUM_POS = 77
BATCH = 1024
SEQ = 77
TOTAL = BATCH * SEQ
NUM_WORKERS = 32
NPIECE = 2
PIECE = TOTAL // NPIECE
ROWS_PER_WORKER = PIECE // NUM_WORKERS
CHUNK = 56
NCHUNKS = ROWS_PER_WORKER // CHUNK
NBUF = 3


def _gather_body(ids_hbm, table_hbm, out_hbm,
                 i0, i1, i2, b0, b1, b2,
                 s0, s1, s2, g0, g1, g2, w0, w1, w2):
  ibufs = (i0, i1, i2)
  isems = (s0, s1, s2)
  bufs = (b0, b1, b2)
  gsems = (g0, g1, g2)
  wsems = (w0, w1, w2)
  num_cores = 2
  wid = lax.axis_index("s") * num_cores + lax.axis_index("c")
  base = wid * ROWS_PER_WORKER

  def i_start(c, b):
    pltpu.async_copy(
        ids_hbm.at[pl.ds(base + c * CHUNK, CHUNK)], ibufs[b], isems[b])

  def i_wait(b):
    pltpu.make_async_copy(
        ids_hbm.at[pl.ds(0, CHUNK)], ibufs[b], isems[b]).wait()

  def g_start(b):
    pltpu.async_copy(table_hbm.at[ibufs[b]], bufs[b], gsems[b])

  def g_wait(b):
    pltpu.make_async_copy(table_hbm.at[ibufs[b]], bufs[b], gsems[b]).wait()

  def w_start(c, b):
    pltpu.async_copy(
        bufs[b], out_hbm.at[pl.ds(base + c * CHUNK, CHUNK)], wsems[b])

  def w_wait(b):
    pltpu.make_async_copy(
        bufs[b], out_hbm.at[pl.ds(0, CHUNK)], wsems[b]).wait()

  i_start(0, 0)
  i_start(1, 1)
  i_wait(0)
  g_start(0)
  i_wait(1)
  g_start(1)

  def triple(t, carry):
    for b in range(NBUF):
      c = 3 * t + b

      @pl.when(c < NCHUNKS)
      def _step():
        g_wait(b)
        w_start(c, b)

        @pl.when(c >= 1)
        def _free():
          w_wait((b + 2) % NBUF)

        @pl.when(c + 2 < NCHUNKS)
        def _next():
          b2 = (b + 2) % NBUF
          i_start(c + 2, b2)
          i_wait(b2)
          g_start(b2)

    return carry

  lax.fori_loop(0, (NCHUNKS + NBUF) // NBUF, triple, 0)
  w_wait((NCHUNKS - 1) % NBUF)


def _add_body(gathered_ref, poscyc_ref, out_ref):
  out_ref[...] = gathered_ref[...] + poscyc_ref[...]


@jax.jit
def kernel(input_ids, token_embedding, position_embedding):
  ids_flat = input_ids.astype(jnp.int32).reshape(TOTAL)

  mesh = plsc.VectorSubcoreMesh(core_axis_name="c", subcore_axis_name="s")
  gather = pl.kernel(
      _gather_body,
      out_type=jax.ShapeDtypeStruct((PIECE, EMBED), jnp.float32),
      mesh=mesh,
      scratch_types=[
          pltpu.VMEM((CHUNK,), jnp.int32),
          pltpu.VMEM((CHUNK,), jnp.int32),
          pltpu.VMEM((CHUNK,), jnp.int32),
          pltpu.VMEM((CHUNK, EMBED), jnp.float32),
          pltpu.VMEM((CHUNK, EMBED), jnp.float32),
          pltpu.VMEM((CHUNK, EMBED), jnp.float32),
          pltpu.SemaphoreType.DMA,
          pltpu.SemaphoreType.DMA,
          pltpu.SemaphoreType.DMA,
          pltpu.SemaphoreType.DMA,
          pltpu.SemaphoreType.DMA,
          pltpu.SemaphoreType.DMA,
          pltpu.SemaphoreType.DMA,
          pltpu.SemaphoreType.DMA,
          pltpu.SemaphoreType.DMA,
      ],
  )
  poscyc = jnp.tile(position_embedding, (ROWS_PER_WORKER // NUM_POS, 1))
  add = pl.pallas_call(
      _add_body,
      out_shape=jax.ShapeDtypeStruct((PIECE, EMBED), jnp.float32),
      grid=(PIECE // ROWS_PER_WORKER,),
      in_specs=[
          pl.BlockSpec((ROWS_PER_WORKER, EMBED), lambda i: (i, 0)),
          pl.BlockSpec((ROWS_PER_WORKER, EMBED), lambda i: (0, 0)),
      ],
      out_specs=pl.BlockSpec((ROWS_PER_WORKER, EMBED), lambda i: (i, 0)),
  )

  pieces = []
  for k in range(NPIECE):
    g = gather(ids_flat[k * PIECE:(k + 1) * PIECE], token_embedding)
    pieces.append(add(g, poscyc))
  out = jnp.concatenate(pieces, axis=0)
  return out.reshape(BATCH, SEQ, EMBED)


# speedup vs baseline: 1.2113x; 1.2113x over previous
import functools

import jax
import jax.numpy as jnp
from jax import lax
from jax.experimental import pallas as pl
from jax.experimental.pallas import tpu as pltpu
from jax.experimental.pallas import tpu_sc as plsc

VOCAB = 49408
EMBED = 512
NUM_POS = 77
BATCH = 1024
SEQ = 77
TOTAL = BATCH * SEQ
NUM_WORKERS = 32
ROWS_PER_WORKER = TOTAL // NUM_WORKERS
CHUNK = 56
NCHUNKS = ROWS_PER_WORKER // CHUNK
NBUF = 3


def _gather_body(ids_hbm, table_hbm, out_hbm,
                 i0, i1, i2, b0, b1, b2,
                 s0, s1, s2, g0, g1, g2, w0, w1, w2):
  ibufs = (i0, i1, i2)
  isems = (s0, s1, s2)
  bufs = (b0, b1, b2)
  gsems = (g0, g1, g2)
  wsems = (w0, w1, w2)
  num_cores = 2
  wid = lax.axis_index("s") * num_cores + lax.axis_index("c")
  base = wid * ROWS_PER_WORKER

  def i_start(c, b):
    pltpu.async_copy(
        ids_hbm.at[pl.ds(base + c * CHUNK, CHUNK)], ibufs[b], isems[b])

  def i_wait(b):
    pltpu.make_async_copy(
        ids_hbm.at[pl.ds(0, CHUNK)], ibufs[b], isems[b]).wait()

  def g_start(b):
    pltpu.async_copy(table_hbm.at[ibufs[b]], bufs[b], gsems[b])

  def g_wait(b):
    pltpu.make_async_copy(table_hbm.at[ibufs[b]], bufs[b], gsems[b]).wait()

  def w_start(c, b):
    pltpu.async_copy(
        bufs[b], out_hbm.at[pl.ds(base + c * CHUNK, CHUNK)], wsems[b])

  def w_wait(b):
    pltpu.make_async_copy(
        bufs[b], out_hbm.at[pl.ds(0, CHUNK)], wsems[b]).wait()

  i_start(0, 0)
  i_start(1, 1)
  i_start(2, 2)
  i_wait(0)
  g_start(0)
  i_wait(1)
  g_start(1)

  def triple(t, carry):
    for b in range(NBUF):
      c = 3 * t + b

      @pl.when(c < NCHUNKS)
      def _step():
        g_wait(b)
        w_start(c, b)

        @pl.when(c >= 1)
        def _free():
          w_wait((b + 2) % NBUF)

        @pl.when(c + 3 < NCHUNKS)
        def _pref():
          i_start(c + 3, b)

        @pl.when(c + 2 < NCHUNKS)
        def _next():
          b2 = (b + 2) % NBUF
          i_wait(b2)
          g_start(b2)

    return carry

  lax.fori_loop(0, (NCHUNKS + NBUF) // NBUF, triple, 0)
  w_wait((NCHUNKS - 1) % NBUF)


def _add_body(gathered_ref, poscyc_ref, out_ref):
  out_ref[...] = gathered_ref[...] + poscyc_ref[...]


@jax.jit
def kernel(input_ids, token_embedding, position_embedding):
  ids_flat = input_ids.astype(jnp.int32).reshape(TOTAL)

  mesh = plsc.VectorSubcoreMesh(core_axis_name="c", subcore_axis_name="s")
  gather = pl.kernel(
      _gather_body,
      out_type=jax.ShapeDtypeStruct((TOTAL, EMBED), jnp.float32),
      mesh=mesh,
      scratch_types=[
          pltpu.VMEM((CHUNK,), jnp.int32),
          pltpu.VMEM((CHUNK,), jnp.int32),
          pltpu.VMEM((CHUNK,), jnp.int32),
          pltpu.VMEM((CHUNK, EMBED), jnp.float32),
          pltpu.VMEM((CHUNK, EMBED), jnp.float32),
          pltpu.VMEM((CHUNK, EMBED), jnp.float32),
          pltpu.SemaphoreType.DMA,
          pltpu.SemaphoreType.DMA,
          pltpu.SemaphoreType.DMA,
          pltpu.SemaphoreType.DMA,
          pltpu.SemaphoreType.DMA,
          pltpu.SemaphoreType.DMA,
          pltpu.SemaphoreType.DMA,
          pltpu.SemaphoreType.DMA,
          pltpu.SemaphoreType.DMA,
      ],
  )
  gathered = gather(ids_flat, token_embedding)

  poscyc = jnp.tile(position_embedding, (ROWS_PER_WORKER // NUM_POS, 1))
  out = pl.pallas_call(
      _add_body,
      out_shape=jax.ShapeDtypeStruct((TOTAL, EMBED), jnp.float32),
      grid=(TOTAL // ROWS_PER_WORKER,),
      in_specs=[
          pl.BlockSpec((ROWS_PER_WORKER, EMBED), lambda i: (i, 0)),
          pl.BlockSpec((ROWS_PER_WORKER, EMBED), lambda i: (0, 0)),
      ],
      out_specs=pl.BlockSpec((ROWS_PER_WORKER, EMBED), lambda i: (i, 0)),
  )(gathered, poscyc)
  return out.reshape(BATCH, SEQ, EMBED)
